# Initial kernel scaffold; baseline (speedup 1.0000x reference)
import jax
import jax.numpy as jnp
from jax.experimental import pallas as pl


def kernel(position_ids, hidden_states, ln1_w, ln2_w, qkv_w, qkv_b, o_w, o_b, sinks, router_w, router_b, w1, b1, w2, b2):
    raise NotImplementedError("write your pallas kernel here")


# speedup vs baseline: 4.2920x; 4.2920x over previous
import functools
import math

import jax
import jax.numpy as jnp
import numpy as np
from jax.experimental import pallas as pl
from jax.experimental.pallas import tpu as pltpu

EPS = 1e-5
ROPE_BASE, YARN_FACTOR, YARN_ORIG, BETA_FAST, BETA_SLOW = 150000.0, 32.0, 4096, 32.0, 1.0
ALPHA, LIMIT = 1.702, 7.0

BT = 256
BN = 512


def _yarn_inv_freq_mscale(dh):
    pf = ROPE_BASE ** (np.arange(0, dh, 2, dtype=np.float64) / dh)
    inv_extra = 1.0 / pf
    inv_inter = 1.0 / (YARN_FACTOR * pf)

    def fd(r):
        return (dh * math.log(YARN_ORIG / (r * 2.0 * math.pi))) / (2.0 * math.log(ROPE_BASE))

    low = max(int(math.floor(fd(BETA_FAST))), 0)
    high = min(int(math.ceil(fd(BETA_SLOW))), dh // 2 - 1)
    r = np.arange(dh // 2, dtype=np.float64)
    ramp = np.clip((r - low) / max(high - low, 1), 0.0, 1.0)
    mask = 1.0 - ramp
    inv_freq = inv_inter * (1.0 - mask) + inv_extra * mask
    mscale = 0.1 * math.log(YARN_FACTOR) + 1.0
    return np.asarray(inv_freq, np.float32), np.float32(mscale)


def _swap_halves(x):
    h = x.shape[-1] // 2
    return jnp.concatenate([x[:, h:], x[:, :h]], axis=1)


def _qkv_body(ln1_ref, x_ref, w_ref, b_ref, o_ref):
    x = x_ref[:]
    nr = jax.lax.rsqrt(jnp.mean(x * x, axis=1, keepdims=True) + EPS)
    xn = (x * nr * ln1_ref[:]).astype(jnp.bfloat16)
    o_ref[:] = jnp.dot(xn, w_ref[:].astype(jnp.bfloat16),
                       preferred_element_type=jnp.float32) + b_ref[:]


def _attn_body(sink_ref, cosq_ref, sinq_ref, cosf_ref, sinf_ref,
               q_ref, k_ref, v_ref, o_ref, s_scr, *, bq, bk, dh):
    h_id = pl.program_id(0)
    i = pl.program_id(1)
    scale = dh ** -0.5
    q = q_ref[:]
    qr = (q * cosq_ref[:] + _swap_halves(q) * sinq_ref[:]).astype(jnp.bfloat16)
    sink = sink_ref[0, h_id]

    def p1(j, m):
        kb = k_ref[pl.ds(j * bk, bk), :]
        cb = cosf_ref[pl.ds(j * bk, bk), :]
        sb = sinf_ref[pl.ds(j * bk, bk), :]
        kr = (kb * cb + _swap_halves(kb) * sb).astype(jnp.bfloat16)
        s = jax.lax.dot_general(qr, kr, (((1,), (1,)), ((), ())),
                                preferred_element_type=jnp.float32) * scale
        rowp = i * bq + jax.lax.broadcasted_iota(jnp.int32, (bq, bk), 0)
        colp = j * bk + jax.lax.broadcasted_iota(jnp.int32, (bq, bk), 1)
        s = jnp.where(colp <= rowp, s, -1e30)
        s_scr[:, pl.ds(j * bk, bk)] = s
        return jnp.maximum(m, jnp.max(s, axis=1))

    m = jax.lax.fori_loop(0, i + 1, p1, jnp.full((bq,), sink, jnp.float32))

    def p2(j, l):
        e = jnp.exp(s_scr[:, pl.ds(j * bk, bk)] - m[:, None])
        s_scr[:, pl.ds(j * bk, bk)] = e
        return l + jnp.sum(e, axis=1)

    l = jax.lax.fori_loop(0, i + 1, p2, jnp.exp(sink - m))

    def p3(j, acc):
        p = (s_scr[:, pl.ds(j * bk, bk)] / l[:, None]).astype(jnp.bfloat16)
        vb = v_ref[pl.ds(j * bk, bk), :].astype(jnp.bfloat16)
        return acc + jnp.dot(p, vb, preferred_element_type=jnp.float32)

    acc = jax.lax.fori_loop(0, i + 1, p3, jnp.zeros((bq, dh), jnp.float32))
    o_ref[:] = acc


def _oproj_body(attn_ref, ow_ref, ob_ref, res_ref, ln2_ref, rw_ref, rb_ref,
                h_ref, x2_ref, wf_ref):
    a = attn_ref[:].astype(jnp.bfloat16)
    hcur = jnp.dot(a, ow_ref[:].astype(jnp.bfloat16),
                   preferred_element_type=jnp.float32) + ob_ref[:] + res_ref[:]
    h_ref[:] = hcur
    nr = jax.lax.rsqrt(jnp.mean(hcur * hcur, axis=1, keepdims=True) + EPS)
    x2 = hcur * nr * ln2_ref[:]
    x2_ref[:] = x2
    g = jnp.dot(x2.astype(jnp.bfloat16), rw_ref[:].astype(jnp.bfloat16),
                preferred_element_type=jnp.float32) + rb_ref[:]
    bt, lanes = g.shape
    idx = jax.lax.broadcasted_iota(jnp.int32, (bt, lanes), 1)
    t1v = jnp.max(g, axis=1, keepdims=True)
    t1i = jnp.min(jnp.where(g == t1v, idx, lanes - 1), axis=1, keepdims=True)
    g2 = jnp.where(idx == t1i, -1e30, g)
    t2v = jnp.max(g2, axis=1, keepdims=True)
    t2i = jnp.min(jnp.where(g2 == t2v, idx, lanes - 1), axis=1, keepdims=True)
    r1 = jax.nn.sigmoid(t1v - t2v)
    r2 = 1.0 - r1
    wf_ref[:] = jnp.where(idx == t1i, r1, 0.0) + jnp.where(idx == t2i, r2, 0.0)


def _moe1_body(x2_ref, w1_ref, b1_ref, se_ref, so_ref, act_ref):
    gu = jnp.dot(x2_ref[:].astype(jnp.bfloat16), w1_ref[:].astype(jnp.bfloat16),
                 preferred_element_type=jnp.float32) + b1_ref[:]
    gub = gu.astype(jnp.bfloat16)
    gate = jnp.dot(gub, se_ref[:], preferred_element_type=jnp.float32)
    up = jnp.dot(gub, so_ref[:], preferred_element_type=jnp.float32)
    gate = jnp.minimum(gate, LIMIT)
    up = jnp.clip(up, -LIMIT, LIMIT)
    glu = gate * jax.nn.sigmoid(gate * ALPHA)
    act_ref[:] = (up + 1.0) * glu


def _moe2_body(act_ref, w2_ref, b2_ref, wf_ref, o_ref, *, e):
    y = jnp.dot(act_ref[:].astype(jnp.bfloat16), w2_ref[:].astype(jnp.bfloat16),
                preferred_element_type=jnp.float32) + b2_ref[:]
    we = wf_ref[:, e:e + 1]
    o_ref[:] = y * we


def kernel(position_ids, hidden_states, ln1_w, ln2_w, qkv_w, qkv_b, o_w, o_b,
           sinks, router_w, router_b, w1, b1, w2, b2):
    T, D = hidden_states.shape
    NQKV = qkv_w.shape[1]
    E = router_w.shape[1]
    DH = 128
    H = o_w.shape[0] // DH
    KVH = (NQKV - H * DH) // (2 * DH)
    FF = w2.shape[1]
    f32 = jnp.float32

    inv_freq, mscale = _yarn_inv_freq_mscale(DH)
    freqs = position_ids.astype(f32)[:, None] * jnp.asarray(inv_freq)[None, :]
    cos = jnp.cos(freqs) * mscale
    sin = jnp.sin(freqs) * mscale
    cos_t = jnp.concatenate([cos, cos], axis=1)
    sin_t = jnp.concatenate([-sin, sin], axis=1)

    nT = T // BT

    ln1_2d = ln1_w.reshape(1, D)
    qkv_b2 = qkv_b.reshape(1, NQKV)
    qkv = pl.pallas_call(
        _qkv_body,
        grid=(nT, NQKV // BN),
        in_specs=[
            pl.BlockSpec((1, D), lambda i, j: (0, 0)),
            pl.BlockSpec((BT, D), lambda i, j: (i, 0)),
            pl.BlockSpec((D, BN), lambda i, j: (0, j)),
            pl.BlockSpec((1, BN), lambda i, j: (0, j)),
        ],
        out_specs=pl.BlockSpec((BT, BN), lambda i, j: (i, j)),
        out_shape=jax.ShapeDtypeStruct((T, NQKV), f32),
    )(ln1_2d, hidden_states, qkv_w, qkv_b2)

    q_mat = qkv[:, : H * DH]
    k_mat = qkv[:, H * DH:(H + KVH) * DH]
    v_mat = qkv[:, (H + KVH) * DH:]
    rep = H // KVH

    attn = pl.pallas_call(
        functools.partial(_attn_body, bq=BT, bk=BT, dh=DH),
        grid=(H, nT),
        in_specs=[
            pl.BlockSpec(memory_space=pltpu.SMEM),
            pl.BlockSpec((BT, DH), lambda h, i: (i, 0)),
            pl.BlockSpec((BT, DH), lambda h, i: (i, 0)),
            pl.BlockSpec((T, DH), lambda h, i: (0, 0)),
            pl.BlockSpec((T, DH), lambda h, i: (0, 0)),
            pl.BlockSpec((BT, DH), lambda h, i: (i, h)),
            pl.BlockSpec((T, DH), lambda h, i: (0, h // rep)),
            pl.BlockSpec((T, DH), lambda h, i: (0, h // rep)),
        ],
        out_specs=pl.BlockSpec((BT, DH), lambda h, i: (i, h)),
        out_shape=jax.ShapeDtypeStruct((T, H * DH), f32),
        scratch_shapes=[pltpu.VMEM((BT, T), jnp.float32)],
    )(sinks.reshape(1, H), cos_t, sin_t, cos_t, sin_t, q_mat, k_mat, v_mat)

    LANES = 128
    rw_pad = jnp.zeros((D, LANES), f32).at[:, :E].set(router_w)
    rb_pad = jnp.full((1, LANES), -1e30, f32).at[0, :E].set(router_b)
    h_out, x2, wf = pl.pallas_call(
        _oproj_body,
        grid=(nT,),
        in_specs=[
            pl.BlockSpec((BT, H * DH), lambda i: (i, 0)),
            pl.BlockSpec((H * DH, D), lambda i: (0, 0)),
            pl.BlockSpec((1, D), lambda i: (0, 0)),
            pl.BlockSpec((BT, D), lambda i: (i, 0)),
            pl.BlockSpec((1, D), lambda i: (0, 0)),
            pl.BlockSpec((D, LANES), lambda i: (0, 0)),
            pl.BlockSpec((1, LANES), lambda i: (0, 0)),
        ],
        out_specs=[
            pl.BlockSpec((BT, D), lambda i: (i, 0)),
            pl.BlockSpec((BT, D), lambda i: (i, 0)),
            pl.BlockSpec((BT, LANES), lambda i: (i, 0)),
        ],
        out_shape=[
            jax.ShapeDtypeStruct((T, D), f32),
            jax.ShapeDtypeStruct((T, D), f32),
            jax.ShapeDtypeStruct((T, LANES), f32),
        ],
    )(attn, o_w, o_b.reshape(1, D), hidden_states, ln2_w.reshape(1, D),
      rw_pad, rb_pad)

    sel = np.zeros((BN, BN // 2), np.float32)
    sel[np.arange(0, BN, 2), np.arange(BN // 2)] = 1.0
    s_even = jnp.asarray(sel, jnp.bfloat16)
    sel_o = np.zeros((BN, BN // 2), np.float32)
    sel_o[np.arange(1, BN, 2), np.arange(BN // 2)] = 1.0
    s_odd = jnp.asarray(sel_o, jnp.bfloat16)

    moe_terms = []
    for e in range(E):
        act = pl.pallas_call(
            _moe1_body,
            grid=(nT, (2 * FF) // BN),
            in_specs=[
                pl.BlockSpec((BT, D), lambda i, j: (i, 0)),
                pl.BlockSpec((D, BN), lambda i, j: (0, j)),
                pl.BlockSpec((1, BN), lambda i, j: (0, j)),
                pl.BlockSpec((BN, BN // 2), lambda i, j: (0, 0)),
                pl.BlockSpec((BN, BN // 2), lambda i, j: (0, 0)),
            ],
            out_specs=pl.BlockSpec((BT, BN // 2), lambda i, j: (i, j)),
            out_shape=jax.ShapeDtypeStruct((T, FF), f32),
        )(x2, w1[e], b1[e].reshape(1, 2 * FF), s_even, s_odd)
        ye = pl.pallas_call(
            functools.partial(_moe2_body, e=e),
            grid=(nT, D // BN),
            in_specs=[
                pl.BlockSpec((BT, FF), lambda i, j: (i, 0)),
                pl.BlockSpec((FF, BN), lambda i, j: (0, j)),
                pl.BlockSpec((1, BN), lambda i, j: (0, j)),
                pl.BlockSpec((BT, LANES), lambda i, j: (i, 0)),
            ],
            out_specs=pl.BlockSpec((BT, BN), lambda i, j: (i, j)),
            out_shape=jax.ShapeDtypeStruct((T, D), f32),
        )(act, w2[e], b2[e].reshape(1, D), wf)
        moe_terms.append(ye)

    out = h_out
    for ye in moe_terms:
        out = out + ye
    return (out, 0)


# speedup vs baseline: 6.6610x; 1.5520x over previous
import functools
import math

import jax
import jax.numpy as jnp
import numpy as np
from jax.experimental import pallas as pl
from jax.experimental.pallas import tpu as pltpu

EPS = 1e-5
ROPE_BASE, YARN_FACTOR, YARN_ORIG, BETA_FAST, BETA_SLOW = 150000.0, 32.0, 4096, 32.0, 1.0
ALPHA, LIMIT = 1.702, 7.0

BT = 256
BN = 512


def _yarn_inv_freq_mscale(dh):
    pf = ROPE_BASE ** (np.arange(0, dh, 2, dtype=np.float64) / dh)
    inv_extra = 1.0 / pf
    inv_inter = 1.0 / (YARN_FACTOR * pf)

    def fd(r):
        return (dh * math.log(YARN_ORIG / (r * 2.0 * math.pi))) / (2.0 * math.log(ROPE_BASE))

    low = max(int(math.floor(fd(BETA_FAST))), 0)
    high = min(int(math.ceil(fd(BETA_SLOW))), dh // 2 - 1)
    r = np.arange(dh // 2, dtype=np.float64)
    ramp = np.clip((r - low) / max(high - low, 1), 0.0, 1.0)
    mask = 1.0 - ramp
    inv_freq = inv_inter * (1.0 - mask) + inv_extra * mask
    mscale = 0.1 * math.log(YARN_FACTOR) + 1.0
    return np.asarray(inv_freq, np.float32), np.float32(mscale)


def _swap_halves(x):
    h = x.shape[-1] // 2
    return jnp.concatenate([x[:, h:], x[:, :h]], axis=1)


def _qkv_body(ln1_ref, x_ref, w_ref, b_ref, o_ref):
    x = x_ref[:]
    nr = jax.lax.rsqrt(jnp.mean(x * x, axis=1, keepdims=True) + EPS)
    xn = (x * nr * ln1_ref[:]).astype(jnp.bfloat16)
    o_ref[:] = jnp.dot(xn, w_ref[:].astype(jnp.bfloat16),
                       preferred_element_type=jnp.float32) + b_ref[:]


def _attn_body(sink_ref, cosq_ref, sinq_ref, cosf_ref, sinf_ref,
               q_ref, k_ref, v_ref, o_ref, s_scr, *, bq, bk, dh):
    h_id = pl.program_id(0)
    i = pl.program_id(1)
    scale = dh ** -0.5
    q = q_ref[:]
    qr = (q * cosq_ref[:] + _swap_halves(q) * sinq_ref[:]).astype(jnp.bfloat16)
    sink = sink_ref[0, h_id]

    def p1(j, m):
        kb = k_ref[pl.ds(j * bk, bk), :]
        cb = cosf_ref[pl.ds(j * bk, bk), :]
        sb = sinf_ref[pl.ds(j * bk, bk), :]
        kr = (kb * cb + _swap_halves(kb) * sb).astype(jnp.bfloat16)
        s = jax.lax.dot_general(qr, kr, (((1,), (1,)), ((), ())),
                                preferred_element_type=jnp.float32) * scale
        rowp = i * bq + jax.lax.broadcasted_iota(jnp.int32, (bq, bk), 0)
        colp = j * bk + jax.lax.broadcasted_iota(jnp.int32, (bq, bk), 1)
        s = jnp.where(colp <= rowp, s, -1e30)
        s_scr[:, pl.ds(j * bk, bk)] = s
        return jnp.maximum(m, jnp.max(s, axis=1))

    m = jax.lax.fori_loop(0, i + 1, p1, jnp.full((bq,), sink, jnp.float32))

    def p2(j, l):
        e = jnp.exp(s_scr[:, pl.ds(j * bk, bk)] - m[:, None])
        s_scr[:, pl.ds(j * bk, bk)] = e
        return l + jnp.sum(e, axis=1)

    l = jax.lax.fori_loop(0, i + 1, p2, jnp.exp(sink - m))

    def p3(j, acc):
        p = (s_scr[:, pl.ds(j * bk, bk)] / l[:, None]).astype(jnp.bfloat16)
        vb = v_ref[pl.ds(j * bk, bk), :].astype(jnp.bfloat16)
        return acc + jnp.dot(p, vb, preferred_element_type=jnp.float32)

    acc = jax.lax.fori_loop(0, i + 1, p3, jnp.zeros((bq, dh), jnp.float32))
    o_ref[:] = acc


def _oproj_body(attn_ref, ow_ref, ob_ref, res_ref, ln2_ref, rw_ref, rb_ref,
                h_ref, x2_ref, wf_ref):
    a = attn_ref[:].astype(jnp.bfloat16)
    hcur = jnp.dot(a, ow_ref[:].astype(jnp.bfloat16),
                   preferred_element_type=jnp.float32) + ob_ref[:] + res_ref[:]
    h_ref[:] = hcur
    nr = jax.lax.rsqrt(jnp.mean(hcur * hcur, axis=1, keepdims=True) + EPS)
    x2 = hcur * nr * ln2_ref[:]
    x2_ref[:] = x2
    g = jnp.dot(x2.astype(jnp.bfloat16), rw_ref[:].astype(jnp.bfloat16),
                preferred_element_type=jnp.float32) + rb_ref[:]
    bt, lanes = g.shape
    idx = jax.lax.broadcasted_iota(jnp.int32, (bt, lanes), 1)
    t1v = jnp.max(g, axis=1, keepdims=True)
    t1i = jnp.min(jnp.where(g == t1v, idx, lanes - 1), axis=1, keepdims=True)
    g2 = jnp.where(idx == t1i, -1e30, g)
    t2v = jnp.max(g2, axis=1, keepdims=True)
    t2i = jnp.min(jnp.where(g2 == t2v, idx, lanes - 1), axis=1, keepdims=True)
    r1 = jax.nn.sigmoid(t1v - t2v)
    r2 = 1.0 - r1
    wf_ref[:] = jnp.where(idx == t1i, r1, 0.0) + jnp.where(idx == t2i, r2, 0.0)


def _moe1_body(x2_ref, w1_ref, b1_ref, se_ref, so_ref, act_ref):
    gu = jnp.dot(x2_ref[:].astype(jnp.bfloat16), w1_ref[:].astype(jnp.bfloat16),
                 preferred_element_type=jnp.float32) + b1_ref[:]
    gub = gu.astype(jnp.bfloat16)
    gate = jnp.dot(gub, se_ref[:], preferred_element_type=jnp.float32)
    up = jnp.dot(gub, so_ref[:], preferred_element_type=jnp.float32)
    gate = jnp.minimum(gate, LIMIT)
    up = jnp.clip(up, -LIMIT, LIMIT)
    glu = gate * jax.nn.sigmoid(gate * ALPHA)
    act_ref[:] = ((up + 1.0) * glu).astype(jnp.bfloat16)


def _moe2_body(acc_ref, act_ref, w2_ref, b2_ref, wf_ref, o_ref, *, e):
    y = jnp.dot(act_ref[:], w2_ref[:].astype(jnp.bfloat16),
                preferred_element_type=jnp.float32) + b2_ref[:]
    we = wf_ref[:, e:e + 1]
    o_ref[:] = acc_ref[:] + y * we


def kernel(position_ids, hidden_states, ln1_w, ln2_w, qkv_w, qkv_b, o_w, o_b,
           sinks, router_w, router_b, w1, b1, w2, b2):
    T, D = hidden_states.shape
    NQKV = qkv_w.shape[1]
    E = router_w.shape[1]
    DH = 128
    H = o_w.shape[0] // DH
    KVH = (NQKV - H * DH) // (2 * DH)
    FF = w2.shape[1]
    f32 = jnp.float32

    inv_freq, mscale = _yarn_inv_freq_mscale(DH)
    freqs = position_ids.astype(f32)[:, None] * jnp.asarray(inv_freq)[None, :]
    cos = jnp.cos(freqs) * mscale
    sin = jnp.sin(freqs) * mscale
    cos_t = jnp.concatenate([cos, cos], axis=1)
    sin_t = jnp.concatenate([-sin, sin], axis=1)

    nT = T // BT

    ln1_2d = ln1_w.reshape(1, D)
    qkv_b2 = qkv_b.reshape(1, NQKV)
    qkv = pl.pallas_call(
        _qkv_body,
        grid=(NQKV // BN,),
        in_specs=[
            pl.BlockSpec((1, D), lambda j: (0, 0)),
            pl.BlockSpec((T, D), lambda j: (0, 0)),
            pl.BlockSpec((D, BN), lambda j: (0, j)),
            pl.BlockSpec((1, BN), lambda j: (0, j)),
        ],
        out_specs=pl.BlockSpec((T, BN), lambda j: (0, j)),
        out_shape=jax.ShapeDtypeStruct((T, NQKV), f32),
    )(ln1_2d, hidden_states, qkv_w, qkv_b2)

    q_mat = qkv[:, : H * DH]
    k_mat = qkv[:, H * DH:(H + KVH) * DH]
    v_mat = qkv[:, (H + KVH) * DH:]
    rep = H // KVH

    attn = pl.pallas_call(
        functools.partial(_attn_body, bq=BT, bk=BT, dh=DH),
        grid=(H, nT),
        in_specs=[
            pl.BlockSpec(memory_space=pltpu.SMEM),
            pl.BlockSpec((BT, DH), lambda h, i: (i, 0)),
            pl.BlockSpec((BT, DH), lambda h, i: (i, 0)),
            pl.BlockSpec((T, DH), lambda h, i: (0, 0)),
            pl.BlockSpec((T, DH), lambda h, i: (0, 0)),
            pl.BlockSpec((BT, DH), lambda h, i: (i, h)),
            pl.BlockSpec((T, DH), lambda h, i: (0, h // rep)),
            pl.BlockSpec((T, DH), lambda h, i: (0, h // rep)),
        ],
        out_specs=pl.BlockSpec((BT, DH), lambda h, i: (i, h)),
        out_shape=jax.ShapeDtypeStruct((T, H * DH), f32),
        scratch_shapes=[pltpu.VMEM((BT, T), jnp.float32)],
    )(sinks.reshape(1, H), cos_t, sin_t, cos_t, sin_t, q_mat, k_mat, v_mat)

    LANES = 128
    rw_pad = jnp.zeros((D, LANES), f32).at[:, :E].set(router_w)
    rb_pad = jnp.full((1, LANES), -1e30, f32).at[0, :E].set(router_b)
    h_out, x2, wf = pl.pallas_call(
        _oproj_body,
        grid=(nT,),
        in_specs=[
            pl.BlockSpec((BT, H * DH), lambda i: (i, 0)),
            pl.BlockSpec((H * DH, D), lambda i: (0, 0)),
            pl.BlockSpec((1, D), lambda i: (0, 0)),
            pl.BlockSpec((BT, D), lambda i: (i, 0)),
            pl.BlockSpec((1, D), lambda i: (0, 0)),
            pl.BlockSpec((D, LANES), lambda i: (0, 0)),
            pl.BlockSpec((1, LANES), lambda i: (0, 0)),
        ],
        out_specs=[
            pl.BlockSpec((BT, D), lambda i: (i, 0)),
            pl.BlockSpec((BT, D), lambda i: (i, 0)),
            pl.BlockSpec((BT, LANES), lambda i: (i, 0)),
        ],
        out_shape=[
            jax.ShapeDtypeStruct((T, D), f32),
            jax.ShapeDtypeStruct((T, D), f32),
            jax.ShapeDtypeStruct((T, LANES), f32),
        ],
    )(attn, o_w, o_b.reshape(1, D), hidden_states, ln2_w.reshape(1, D),
      rw_pad, rb_pad)

    sel = np.zeros((BN, BN // 2), np.float32)
    sel[np.arange(0, BN, 2), np.arange(BN // 2)] = 1.0
    s_even = jnp.asarray(sel, jnp.bfloat16)
    sel_o = np.zeros((BN, BN // 2), np.float32)
    sel_o[np.arange(1, BN, 2), np.arange(BN // 2)] = 1.0
    s_odd = jnp.asarray(sel_o, jnp.bfloat16)

    x2b = x2
    out = h_out
    for e in range(E):
        act = pl.pallas_call(
            _moe1_body,
            grid=((2 * FF) // BN,),
            in_specs=[
                pl.BlockSpec((T, D), lambda j: (0, 0)),
                pl.BlockSpec((D, BN), lambda j: (0, j)),
                pl.BlockSpec((1, BN), lambda j: (0, j)),
                pl.BlockSpec((BN, BN // 2), lambda j: (0, 0)),
                pl.BlockSpec((BN, BN // 2), lambda j: (0, 0)),
            ],
            out_specs=pl.BlockSpec((T, BN // 2), lambda j: (0, j)),
            out_shape=jax.ShapeDtypeStruct((T, FF), jnp.bfloat16),
        )(x2b, w1[e], b1[e].reshape(1, 2 * FF), s_even, s_odd)
        out = pl.pallas_call(
            functools.partial(_moe2_body, e=e),
            grid=(D // BN,),
            in_specs=[
                pl.BlockSpec((T, BN), lambda j: (0, j)),
                pl.BlockSpec((T, FF), lambda j: (0, 0)),
                pl.BlockSpec((FF, BN), lambda j: (0, j)),
                pl.BlockSpec((1, BN), lambda j: (0, j)),
                pl.BlockSpec((T, LANES), lambda j: (0, 0)),
            ],
            out_specs=pl.BlockSpec((T, BN), lambda j: (0, j)),
            out_shape=jax.ShapeDtypeStruct((T, D), f32),
            input_output_aliases={0: 0},
        )(out, act, w2[e], b2[e].reshape(1, D), wf)
    return (out, 0)
